# baseline (speedup 1.0000x reference)
import functools

import jax
import jax.numpy as jnp
from jax import lax
from jax.experimental import pallas as pl
from jax.experimental.pallas import tpu as pltpu
from jax.experimental.pallas import tpu_sc as plsc

N = 10000
E = 320000
D = 128

NC = 2
NS = 16
L = 128

N_PAD = 10112
ROWS_N = N_PAD // NS

E_PAD = 327680
PAD = E_PAD - E
EDGE_ROWS = E_PAD // L
RPW = EDGE_ROWS // (NC * NS)
R_C0 = 80
R_C1 = 2 * RPW - R_C0

BLK_MID = 1264
BLK_FIN = 1000


_mesh = plsc.VectorSubcoreMesh(core_axis_name="c", subcore_axis_name="s",
                               num_cores=NC, num_subcores=NS)


@functools.partial(
    pl.kernel,
    out_type=jax.ShapeDtypeStruct((NC, N_PAD, D), jnp.float32),
    mesh=_mesh,
    scratch_types=[
        pltpu.VMEM_SHARED((N_PAD, D), jnp.float32),
        pltpu.VMEM((L,), jnp.int32),
        pltpu.VMEM((L,), jnp.int32),
        pltpu.VMEM((L,), jnp.int32),
        pltpu.VMEM((L,), jnp.int32),
        pltpu.VMEM((L, D), jnp.float32),
        pltpu.VMEM((L, D), jnp.float32),
        pltpu.SemaphoreType.DMA,
        pltpu.SemaphoreType.DMA,
        pltpu.SemaphoreType.DMA,
        pltpu.SemaphoreType.DMA,
    ],
)
def _sc_aggregate(h_hbm, src_hbm, dst_hbm, zeros_hbm, out_hbm, accum,
                  sidx0, sidx1, didx0, didx1, rows0, rows1,
                  gsem0, gsem1, isem0, isem1):
    c = lax.axis_index("c")
    s = lax.axis_index("s")
    pltpu.sync_copy(zeros_hbm, accum.at[pl.ds(s * ROWS_N, ROWS_N)])
    plsc.subcore_barrier()

    sets = ((sidx0, didx0, rows0, gsem0, isem0),
            (sidx1, didx1, rows1, gsem1, isem1))

    def serial_loop(base, nrows):
        def body(t, carry):
            pltpu.sync_copy(src_hbm.at[base + t], sidx0)
            pltpu.sync_copy(dst_hbm.at[base + t], didx0)
            pltpu.async_copy(h_hbm.at[sidx0], rows0, gsem0).wait()
            pltpu.sync_copy(rows0, accum.at[didx0], add=True)
            return carry
        lax.fori_loop(0, nrows, body, 0)

    @pl.when(c == 0)
    def _():
        serial_loop(s * R_C0, R_C0)

    @pl.when(c == 1)
    def _():
        serial_loop(NS * R_C0 + s * R_C1, R_C1)

    plsc.subcore_barrier()
    pltpu.sync_copy(accum.at[pl.ds(s * ROWS_N, ROWS_N)],
                    out_hbm.at[c, pl.ds(s * ROWS_N, ROWS_N)])


@functools.partial(
    pl.kernel,
    out_type=jax.ShapeDtypeStruct((NC, N_PAD, D), jnp.float32),
    mesh=_mesh,
    scratch_types=[
        pltpu.VMEM_SHARED((N_PAD, D), jnp.float32),
        pltpu.VMEM((L,), jnp.int32),
        pltpu.VMEM((L,), jnp.int32),
        pltpu.VMEM((L, D), jnp.float32),
        pltpu.SemaphoreType.DMA,
        pltpu.SemaphoreType.DMA,
    ],
)
def _sc_count(dst_hbm, ones_hbm, zeros_hbm, out_hbm, accum,
              didx0, didx1, ones, isem0, isem1):
    c = lax.axis_index("c")
    s = lax.axis_index("s")
    pltpu.sync_copy(zeros_hbm, accum.at[pl.ds(s * ROWS_N, ROWS_N)])
    pltpu.sync_copy(ones_hbm, ones)
    plsc.subcore_barrier()
    base = (c * NS + s) * RPW

    dsets = ((didx0, isem0), (didx1, isem1))

    def step(X, t_pre=None):
        dX, iX = dsets[X]
        pltpu.make_async_copy(dst_hbm.at[0], dX, iX).wait()
        pltpu.sync_copy(ones, accum.at[dX], add=True)
        if t_pre is not None:
            pltpu.async_copy(dst_hbm.at[t_pre], dX, iX)

    pltpu.async_copy(dst_hbm.at[base], didx0, isem0)
    pltpu.async_copy(dst_hbm.at[base + 1], didx1, isem1)

    def pair(k, carry):
        t = 2 * k
        step(0, t_pre=base + t + 2)
        step(1, t_pre=base + t + 3)
        return carry

    lax.fori_loop(0, (RPW - 2) // 2, pair, 0)
    step(0)
    step(1)

    plsc.subcore_barrier()
    pltpu.sync_copy(accum.at[pl.ds(s * ROWS_N, ROWS_N)],
                    out_hbm.at[c, pl.ds(s * ROWS_N, ROWS_N)])


def _dense_body(sums_ref, h_ref, cnt_ref, wl_ref, wr_ref, b_ref, a_ref,
                out_ref):
    rec = 1.0 / jnp.maximum(cnt_ref[:, 0:1], 1.0)
    m = (sums_ref[0] + sums_ref[1]) * rec
    z = (jnp.dot(m, wl_ref[...], preferred_element_type=jnp.float32)
         + jnp.dot(h_ref[...], wr_ref[...], preferred_element_type=jnp.float32)
         + b_ref[0:1, :])
    out_ref[...] = jnp.where(z >= 0, z, a_ref[0:1, :] * z)


def _make_dense(final):
    blk = BLK_FIN if final else BLK_MID
    grid = (N // blk,) if final else (N_PAD // blk,)
    nrows = N if final else N_PAD
    return pl.pallas_call(
        _dense_body,
        grid=grid,
        in_specs=[
            pl.BlockSpec((NC, blk, D), lambda i: (0, i, 0)),
            pl.BlockSpec((blk, D), lambda i: (i, 0)),
            pl.BlockSpec((blk, 8), lambda i: (i, 0)),
            pl.BlockSpec((D, D), lambda i: (0, 0)),
            pl.BlockSpec((D, D), lambda i: (0, 0)),
            pl.BlockSpec((1, D), lambda i: (0, 0)),
            pl.BlockSpec((1, D), lambda i: (0, 0)),
        ],
        out_specs=pl.BlockSpec((blk, D), lambda i: (i, 0)),
        out_shape=jax.ShapeDtypeStruct((nrows, D), jnp.float32),
    )


_dense_mid = _make_dense(False)
_dense_fin = _make_dense(True)


def kernel(x, edge_index, Wl0, Wr0, b0, a0, Wl1, Wr1, b1, a1,
           Wl2, Wr2, b2, a2, Wl3, Wr3, b3, a3):
    ei = edge_index.astype(jnp.int32)
    pad_dst = N + (jnp.arange(PAD, dtype=jnp.int32) % (N_PAD - N))
    src = jnp.concatenate([ei[0], jnp.zeros((PAD,), jnp.int32)]).reshape(EDGE_ROWS, L)
    dst = jnp.concatenate([ei[1], pad_dst]).reshape(EDGE_ROWS, L)
    h = jnp.pad(x, ((0, N_PAD - N), (0, 0)))
    zeros_d = jnp.zeros((ROWS_N, D), jnp.float32)
    ones_d = jnp.ones((L, D), jnp.float32)

    cntp = _sc_count(dst, ones_d, zeros_d)
    cnt = cntp[0, :, 0:8] + cntp[1, :, 0:8]
    h, cnt = lax.optimization_barrier((h, cnt))

    params = [(Wl0, Wr0, b0, a0), (Wl1, Wr1, b1, a1),
              (Wl2, Wr2, b2, a2), (Wl3, Wr3, b3, a3)]
    for i, (Wl, Wr, b, a) in enumerate(params):
        sums = _sc_aggregate(h, src, dst, zeros_d)
        dense = _dense_fin if i == 3 else _dense_mid
        h = dense(sums, h, cnt, Wl, Wr, b.reshape(1, D), a.reshape(1, D))
    return h


# speedup vs baseline: 1.0001x; 1.0001x over previous
import functools

import jax
import jax.numpy as jnp
from jax import lax
from jax.experimental import pallas as pl
from jax.experimental.pallas import tpu as pltpu
from jax.experimental.pallas import tpu_sc as plsc

N = 10000
E = 320000
D = 128

NC = 2
NS = 16
L = 128

N_PAD = 10112
ROWS_N = N_PAD // NS

E_PAD = 327680
PAD = E_PAD - E
EDGE_ROWS = E_PAD // L
RPW = EDGE_ROWS // (NC * NS)
R_C0 = 80
R_C1 = 2 * RPW - R_C0

BLK_MID = 1264
BLK_FIN = 1000


_mesh = plsc.VectorSubcoreMesh(core_axis_name="c", subcore_axis_name="s",
                               num_cores=NC, num_subcores=NS)


@functools.partial(
    pl.kernel,
    out_type=jax.ShapeDtypeStruct((NC, N_PAD, D), jnp.float32),
    mesh=_mesh,
    scratch_types=[
        pltpu.VMEM_SHARED((N_PAD, D), jnp.float32),
        pltpu.VMEM((L,), jnp.int32),
        pltpu.VMEM((L,), jnp.int32),
        pltpu.VMEM((L,), jnp.int32),
        pltpu.VMEM((L,), jnp.int32),
        pltpu.VMEM((L, D), jnp.float32),
        pltpu.VMEM((L, D), jnp.float32),
        pltpu.SemaphoreType.DMA,
        pltpu.SemaphoreType.DMA,
        pltpu.SemaphoreType.DMA,
        pltpu.SemaphoreType.DMA,
    ],
)
def _sc_aggregate(h_hbm, src_hbm, dst_hbm, zeros_hbm, out_hbm, accum,
                  sidx0, sidx1, didx0, didx1, rows0, rows1,
                  gsem0, gsem1, isem0, isem1):
    c = lax.axis_index("c")
    s = lax.axis_index("s")
    pltpu.sync_copy(zeros_hbm, accum.at[pl.ds(s * ROWS_N, ROWS_N)])
    plsc.subcore_barrier()

    base = (c * NS + s) * RPW

    def body(t, carry):
        pltpu.sync_copy(src_hbm.at[base + t], sidx0)
        pltpu.sync_copy(dst_hbm.at[base + t], didx0)
        pltpu.async_copy(h_hbm.at[sidx0], rows0, gsem0).wait()
        pltpu.sync_copy(rows0, accum.at[didx0], add=True)
        return carry

    lax.fori_loop(0, RPW, body, 0)
    plsc.subcore_barrier()
    pltpu.sync_copy(accum.at[pl.ds(s * ROWS_N, ROWS_N)],
                    out_hbm.at[c, pl.ds(s * ROWS_N, ROWS_N)])


@functools.partial(
    pl.kernel,
    out_type=jax.ShapeDtypeStruct((NC, N_PAD, D), jnp.float32),
    mesh=_mesh,
    scratch_types=[
        pltpu.VMEM_SHARED((N_PAD, D), jnp.float32),
        pltpu.VMEM((L,), jnp.int32),
        pltpu.VMEM((L,), jnp.int32),
        pltpu.VMEM((L, D), jnp.float32),
        pltpu.SemaphoreType.DMA,
        pltpu.SemaphoreType.DMA,
    ],
)
def _sc_count(dst_hbm, ones_hbm, zeros_hbm, out_hbm, accum,
              didx0, didx1, ones, isem0, isem1):
    c = lax.axis_index("c")
    s = lax.axis_index("s")
    pltpu.sync_copy(zeros_hbm, accum.at[pl.ds(s * ROWS_N, ROWS_N)])
    pltpu.sync_copy(ones_hbm, ones)
    plsc.subcore_barrier()
    base = (c * NS + s) * RPW

    dsets = ((didx0, isem0), (didx1, isem1))

    def step(X, t_pre=None):
        dX, iX = dsets[X]
        pltpu.make_async_copy(dst_hbm.at[0], dX, iX).wait()
        pltpu.sync_copy(ones, accum.at[dX], add=True)
        if t_pre is not None:
            pltpu.async_copy(dst_hbm.at[t_pre], dX, iX)

    pltpu.async_copy(dst_hbm.at[base], didx0, isem0)
    pltpu.async_copy(dst_hbm.at[base + 1], didx1, isem1)

    def pair(k, carry):
        t = 2 * k
        step(0, t_pre=base + t + 2)
        step(1, t_pre=base + t + 3)
        return carry

    lax.fori_loop(0, (RPW - 2) // 2, pair, 0)
    step(0)
    step(1)

    plsc.subcore_barrier()
    pltpu.sync_copy(accum.at[pl.ds(s * ROWS_N, ROWS_N)],
                    out_hbm.at[c, pl.ds(s * ROWS_N, ROWS_N)])


def _dense_body(sums_ref, h_ref, cnt_ref, wl_ref, wr_ref, b_ref, a_ref,
                out_ref):
    rec = 1.0 / jnp.maximum(cnt_ref[:, 0:1], 1.0)
    m = (sums_ref[0] + sums_ref[1]) * rec
    z = (jnp.dot(m, wl_ref[...], preferred_element_type=jnp.float32)
         + jnp.dot(h_ref[...], wr_ref[...], preferred_element_type=jnp.float32)
         + b_ref[0:1, :])
    out_ref[...] = jnp.where(z >= 0, z, a_ref[0:1, :] * z)


def _make_dense(final):
    blk = BLK_FIN if final else BLK_MID
    grid = (N // blk,) if final else (N_PAD // blk,)
    nrows = N if final else N_PAD
    return pl.pallas_call(
        _dense_body,
        grid=grid,
        in_specs=[
            pl.BlockSpec((NC, blk, D), lambda i: (0, i, 0)),
            pl.BlockSpec((blk, D), lambda i: (i, 0)),
            pl.BlockSpec((blk, 8), lambda i: (i, 0)),
            pl.BlockSpec((D, D), lambda i: (0, 0)),
            pl.BlockSpec((D, D), lambda i: (0, 0)),
            pl.BlockSpec((1, D), lambda i: (0, 0)),
            pl.BlockSpec((1, D), lambda i: (0, 0)),
        ],
        out_specs=pl.BlockSpec((blk, D), lambda i: (i, 0)),
        out_shape=jax.ShapeDtypeStruct((nrows, D), jnp.float32),
    )


_dense_mid = _make_dense(False)
_dense_fin = _make_dense(True)


def kernel(x, edge_index, Wl0, Wr0, b0, a0, Wl1, Wr1, b1, a1,
           Wl2, Wr2, b2, a2, Wl3, Wr3, b3, a3):
    ei = edge_index.astype(jnp.int32)
    src = jnp.concatenate([ei[0], jnp.zeros((PAD,), jnp.int32)]).reshape(EDGE_ROWS, L)
    dst = jnp.concatenate([ei[1], jnp.full((PAD,), N, jnp.int32)]).reshape(EDGE_ROWS, L)
    h = jnp.pad(x, ((0, N_PAD - N), (0, 0)))
    zeros_d = jnp.zeros((ROWS_N, D), jnp.float32)
    ones_d = jnp.ones((L, D), jnp.float32)

    cntp = _sc_count(dst, ones_d, zeros_d)
    cnt = cntp[0, :, 0:8] + cntp[1, :, 0:8]
    h, cnt = lax.optimization_barrier((h, cnt))

    params = [(Wl0, Wr0, b0, a0), (Wl1, Wr1, b1, a1),
              (Wl2, Wr2, b2, a2), (Wl3, Wr3, b3, a3)]
    for i, (Wl, Wr, b, a) in enumerate(params):
        sums = _sc_aggregate(h, src, dst, zeros_d)
        dense = _dense_fin if i == 3 else _dense_mid
        h = dense(sums, h, cnt, Wl, Wr, b.reshape(1, D), a.reshape(1, D))
    return h


# speedup vs baseline: 1.5253x; 1.5252x over previous
import functools

import jax
import jax.numpy as jnp
from jax import lax
from jax.experimental import pallas as pl
from jax.experimental.pallas import tpu as pltpu
from jax.experimental.pallas import tpu_sc as plsc

N = 10000
E = 320000
D = 128

NC = 2
NS = 16
L = 128

N_PAD = 10112
ROWS_N = N_PAD // NS

E_PAD = 323584
PAD = E_PAD - E
EDGE_ROWS = E_PAD // L
RPW = EDGE_ROWS // (NC * NS)
RPS = EDGE_ROWS // NS

BLK_MID = 1264
BLK_FIN = 1000


_mesh = plsc.VectorSubcoreMesh(core_axis_name="c", subcore_axis_name="s",
                               num_cores=NC, num_subcores=NS)


@functools.partial(
    pl.kernel,
    out_type=jax.ShapeDtypeStruct((NC, N_PAD, D), jnp.float32),
    mesh=_mesh,
    scratch_types=[
        pltpu.VMEM_SHARED((N_PAD, D), jnp.float32),
        pltpu.VMEM((L,), jnp.int32),
        pltpu.VMEM((L,), jnp.int32),
        pltpu.VMEM((L, D), jnp.float32),
        pltpu.SemaphoreType.DMA,
    ],
)
def _sc_aggregate(h_hbm, src_hbm, dst_hbm, zeros_hbm, out_hbm,
                  accum, sidx, didx, rows, sem):
    c = lax.axis_index("c")
    s = lax.axis_index("s")
    pltpu.sync_copy(zeros_hbm, accum.at[pl.ds(s * ROWS_N, ROWS_N)])
    plsc.subcore_barrier()
    base = (c * NS + s) * RPW

    def body(t, carry):
        pltpu.sync_copy(src_hbm.at[base + t], sidx)
        pltpu.sync_copy(dst_hbm.at[base + t], didx)
        pltpu.async_copy(h_hbm.at[sidx], rows, sem).wait()
        pltpu.sync_copy(rows, accum.at[didx], add=True)
        return carry

    lax.fori_loop(0, RPW, body, 0)
    plsc.subcore_barrier()
    pltpu.sync_copy(accum.at[pl.ds(s * ROWS_N, ROWS_N)],
                    out_hbm.at[c, pl.ds(s * ROWS_N, ROWS_N)])


@functools.partial(
    pl.kernel,
    out_type=jax.ShapeDtypeStruct((N_PAD, D), jnp.float32),
    mesh=_mesh,
    scratch_types=[
        pltpu.VMEM_SHARED((N_PAD, D), jnp.float32),
        pltpu.VMEM((L,), jnp.int32),
        pltpu.VMEM((L, D), jnp.float32),
        pltpu.SemaphoreType.DMA,
    ],
)
def _sc_count(dst_hbm, ones_hbm, zeros_hbm, out_hbm, accum, didx, ones, sem):
    c = lax.axis_index("c")
    s = lax.axis_index("s")

    @pl.when(c == 0)
    def _():
        pltpu.sync_copy(zeros_hbm, accum.at[pl.ds(s * ROWS_N, ROWS_N)])
        pltpu.sync_copy(ones_hbm, ones)
        plsc.subcore_barrier()
        base = s * RPS

        def body(t, carry):
            pltpu.sync_copy(dst_hbm.at[base + t], didx)
            pltpu.sync_copy(ones, accum.at[didx], add=True)
            return carry

        lax.fori_loop(0, RPS, body, 0)
        plsc.subcore_barrier()
        pltpu.sync_copy(accum.at[pl.ds(s * ROWS_N, ROWS_N)],
                        out_hbm.at[pl.ds(s * ROWS_N, ROWS_N)])


def _dense_body(sums_ref, h_ref, cnt_ref, wl_ref, wr_ref, b_ref, a_ref,
                out_ref):
    rec = 1.0 / jnp.maximum(cnt_ref[:, 0:1], 1.0)
    m = (sums_ref[0] + sums_ref[1]) * rec
    z = (jnp.dot(m, wl_ref[...], preferred_element_type=jnp.float32)
         + jnp.dot(h_ref[...], wr_ref[...], preferred_element_type=jnp.float32)
         + b_ref[0:1, :])
    out_ref[...] = jnp.where(z >= 0, z, a_ref[0:1, :] * z)


def _make_dense(final):
    blk = BLK_FIN if final else BLK_MID
    grid = (N // blk,) if final else (N_PAD // blk,)
    nrows = N if final else N_PAD
    return pl.pallas_call(
        _dense_body,
        grid=grid,
        in_specs=[
            pl.BlockSpec((NC, blk, D), lambda i: (0, i, 0)),
            pl.BlockSpec((blk, D), lambda i: (i, 0)),
            pl.BlockSpec((blk, 8), lambda i: (i, 0)),
            pl.BlockSpec((D, D), lambda i: (0, 0)),
            pl.BlockSpec((D, D), lambda i: (0, 0)),
            pl.BlockSpec((1, D), lambda i: (0, 0)),
            pl.BlockSpec((1, D), lambda i: (0, 0)),
        ],
        out_specs=pl.BlockSpec((blk, D), lambda i: (i, 0)),
        out_shape=jax.ShapeDtypeStruct((nrows, D), jnp.float32),
    )


_dense_mid = _make_dense(False)
_dense_fin = _make_dense(True)


def kernel(x, edge_index, Wl0, Wr0, b0, a0, Wl1, Wr1, b1, a1,
           Wl2, Wr2, b2, a2, Wl3, Wr3, b3, a3):
    ei = edge_index.astype(jnp.int32)
    src = jnp.concatenate([ei[0], jnp.zeros((PAD,), jnp.int32)]).reshape(EDGE_ROWS, L)
    dst = jnp.concatenate([ei[1], jnp.full((PAD,), N, jnp.int32)]).reshape(EDGE_ROWS, L)
    h = jnp.pad(x, ((0, N_PAD - N), (0, 0)))
    zeros_d = jnp.zeros((ROWS_N, D), jnp.float32)
    ones_d = jnp.ones((L, D), jnp.float32)

    cnt = _sc_count(dst, ones_d, zeros_d)[:, 0:8]
    h, cnt = lax.optimization_barrier((h, cnt))

    params = [(Wl0, Wr0, b0, a0), (Wl1, Wr1, b1, a1),
              (Wl2, Wr2, b2, a2), (Wl3, Wr3, b3, a3)]
    for i, (Wl, Wr, b, a) in enumerate(params):
        sums = _sc_aggregate(h, src, dst, zeros_d)
        dense = _dense_fin if i == 3 else _dense_mid
        h = dense(sums, h, cnt, Wl, Wr, b.reshape(1, D), a.reshape(1, D))
    return h
